# Initial kernel scaffold; baseline (speedup 1.0000x reference)
import jax
import jax.numpy as jnp
from jax.experimental import pallas as pl


def kernel(x, edge_attr, edge_index, nn0_W1, nn0_b1, nn0_W2, nn0_b2, root0, bias0, nn1_W1, nn1_b1, nn1_W2, nn1_b2, root1, bias1, ep_W1, ep_b1, ep_W2, ep_b2):
    raise NotImplementedError("write your pallas kernel here")


# speedup vs baseline: 1.7512x; 1.7512x over previous
import functools

import jax
import jax.numpy as jnp
from jax import lax
from jax.experimental import pallas as pl
from jax.experimental.pallas import tpu as pltpu
from jax.experimental.pallas import tpu_sc as plsc

N = 10000
E = 160000
DF = 128
DE = 16
H0 = 8
IH = 10

NC, NS = 2, 16
NW = NC * NS
EPW = 5120
EPAD = NW * EPW
NAG = 10112
RPT = NAG // NS

_f32 = jnp.float32


def _prep_edges_k(ea_ref, w_ref, b_ref, z01_ref, a_ref):
    y = jnp.dot(ea_ref[...], w_ref[...], preferred_element_type=_f32) + b_ref[...]
    be = y.shape[0]
    z01_ref[...] = jnp.maximum(y[:, :20], 0.0)
    a_ref[...] = jnp.concatenate([y[:, 20:], jnp.zeros((be, 6), _f32)], axis=1)


def _node0_k(x_ref, w_ref, b_ref, t_ref, r_ref):
    y = jnp.dot(x_ref[...], w_ref[...], preferred_element_type=_f32)
    be = y.shape[0]
    t_ref[...] = jnp.concatenate([y[:, :88], jnp.zeros((be, 40), _f32)], axis=1)
    r_ref[...] = y[:, 88:] + b_ref[...]


def _comb1_k(agg_ref, xr_ref, w_ref, b_ref, t_ref, r_ref):
    a = agg_ref[0] + agg_ref[1]
    h1 = jnp.maximum(a[:, :8] + a[:, 8:] + xr_ref[...], 0.0)
    y = jnp.dot(h1, w_ref[...], preferred_element_type=_f32)
    be = y.shape[0]
    t_ref[...] = jnp.concatenate([y[:, :110], jnp.zeros((be, 18), _f32)], axis=1)
    r_ref[...] = y[:, 110:] + b_ref[...]


def _combq_k(agg_ref, hr_ref, w_ref, q_ref):
    m = agg_ref[0] + agg_ref[1]
    pre = hr_ref[...]
    for k in range(8):
        pre = pre + m[:, k * 10:k * 10 + 10]
    h2 = jnp.maximum(pre, 0.0)
    be = h2.shape[0]
    h2p = jnp.concatenate([h2, jnp.zeros((be, 6), _f32)], axis=1)
    q = jnp.dot(h2p, w_ref[...], preferred_element_type=_f32)
    q_ref[...] = jnp.concatenate([q, jnp.zeros((be, 112), _f32)], axis=1)


def _finish_k(s_ref, w_ref, o_ref):
    s = jnp.sum(s_ref[...] * w_ref[...], axis=1, keepdims=True)
    o_ref[...] = jnp.broadcast_to(s, s_ref.shape)


def _edge_mm(ea, w, b):
    be, grid = 2048, EPAD // 2048
    return pl.pallas_call(
        _prep_edges_k,
        grid=(grid,),
        in_specs=[
            pl.BlockSpec((be, DE), lambda i: (i, 0)),
            pl.BlockSpec((DE, 30), lambda i: (0, 0)),
            pl.BlockSpec((1, 30), lambda i: (0, 0)),
        ],
        out_specs=[
            pl.BlockSpec((be, 20), lambda i: (i, 0)),
            pl.BlockSpec((be, 16), lambda i: (i, 0)),
        ],
        out_shape=[
            jax.ShapeDtypeStruct((EPAD, 20), _f32),
            jax.ShapeDtypeStruct((EPAD, 16), _f32),
        ],
    )(ea, w, b)


def _node0(x, w, b):
    bn, grid = 1000, N // 1000
    return pl.pallas_call(
        _node0_k,
        grid=(grid,),
        in_specs=[
            pl.BlockSpec((bn, DF), lambda i: (i, 0)),
            pl.BlockSpec((DF, 96), lambda i: (0, 0)),
            pl.BlockSpec((1, 8), lambda i: (0, 0)),
        ],
        out_specs=[
            pl.BlockSpec((bn, 128), lambda i: (i, 0)),
            pl.BlockSpec((bn, 8), lambda i: (i, 0)),
        ],
        out_shape=[
            jax.ShapeDtypeStruct((N, 128), _f32),
            jax.ShapeDtypeStruct((N, 8), _f32),
        ],
    )(x, w, b)


def _comb1(agg, xr, w, b):
    bn, grid = 1000, N // 1000
    return pl.pallas_call(
        _comb1_k,
        grid=(grid,),
        in_specs=[
            pl.BlockSpec((2, bn, 16), lambda i: (0, i, 0)),
            pl.BlockSpec((bn, 8), lambda i: (i, 0)),
            pl.BlockSpec((8, 120), lambda i: (0, 0)),
            pl.BlockSpec((1, 10), lambda i: (0, 0)),
        ],
        out_specs=[
            pl.BlockSpec((bn, 128), lambda i: (i, 0)),
            pl.BlockSpec((bn, 10), lambda i: (i, 0)),
        ],
        out_shape=[
            jax.ShapeDtypeStruct((N, 128), _f32),
            jax.ShapeDtypeStruct((N, 10), _f32),
        ],
    )(agg, xr, w, b)


def _combq(agg, hr, w):
    bn, grid = 1000, N // 1000
    return pl.pallas_call(
        _combq_k,
        grid=(grid,),
        in_specs=[
            pl.BlockSpec((2, bn, 80), lambda i: (0, i, 0)),
            pl.BlockSpec((bn, 10), lambda i: (i, 0)),
            pl.BlockSpec((16, 16), lambda i: (0, 0)),
        ],
        out_specs=pl.BlockSpec((bn, 128), lambda i: (i, 0)),
        out_shape=jax.ShapeDtypeStruct((N, 128), _f32),
    )(agg, hr, w)


def _finish(s2d, w2row):
    be, grid = 2048, EPAD // 2048
    return pl.pallas_call(
        _finish_k,
        grid=(grid,),
        in_specs=[
            pl.BlockSpec((be, 16), lambda i: (i, 0)),
            pl.BlockSpec((1, 16), lambda i: (0, 0)),
        ],
        out_specs=pl.BlockSpec((be, 16), lambda i: (i, 0)),
        out_shape=jax.ShapeDtypeStruct((EPAD, 16), _f32),
    )(s2d, w2row)


def _lane():
    return lax.iota(jnp.int32, 16)


def _zz_segments(col_lo, src_of):
    segs = []
    for l in range(16):
        s = src_of(col_lo + l)
        if not segs or segs[-1][1] != s:
            segs.append((l, s))
    return segs


def _build_zz(segs, zrow, zlane_of_k):
    lane = _lane()

    def val(spec):
        if isinstance(spec, tuple):
            return jnp.full((16,), zrow[zlane_of_k(spec[1])], _f32)
        return jnp.full((16,), spec, _f32)

    zz = val(segs[0][1])
    for b, spec in segs[1:]:
        zz = jnp.where(lane >= b, val(spec), zz)
    return zz


def _make_sc_layer(ch, n_k, n_o, zds, zoff, mw, accumulate):
    nch = EPW // ch
    nj = ch // 128
    nvec = mw // 16 if not accumulate else (n_k * n_o + n_o + 15) // 16

    def src_of(c):
        if c < n_k * n_o:
            return ('z', c // n_o)
        if c < n_k * n_o + n_o:
            return 1.0
        return 0.0

    mesh = plsc.VectorSubcoreMesh(core_axis_name="c", subcore_axis_name="s")

    @functools.partial(
        pl.kernel,
        out_type=jax.ShapeDtypeStruct((NC, NAG, mw), _f32),
        mesh=mesh,
        compiler_params=pltpu.CompilerParams(use_tc_tiling_on_sc=False),
        scratch_types=[
            pltpu.VMEM((nj, 128), jnp.int32),
            pltpu.VMEM((nj, 128), jnp.int32),
            pltpu.VMEM((ch, 20), _f32),
            pltpu.VMEM((ch, 128), _f32),
            pltpu.VMEM((ch, mw), _f32),
            pltpu.VMEM_SHARED((NAG, mw), _f32),
            pltpu.SemaphoreType.DMA,
        ],
    )
    def sc_layer(t_hbm, z_hbm, src_hbm, dst_hbm, agg_hbm,
                 src_v, dst_v, z_v, rows_v, msg_v, agg_sh, sem):
        c = lax.axis_index("c")
        s = lax.axis_index("s")
        wid = c * NS + s

        def zrow_body(r, carry):
            for j in range(mw // 16):
                msg_v[r, pl.ds(j * 16, 16)] = jnp.zeros((16,), _f32)
            return carry

        lax.fori_loop(0, ch, zrow_body, 0)
        done = 0
        while done < RPT:
            n = min(ch, RPT - done)
            pltpu.sync_copy(msg_v.at[pl.ds(0, n)],
                            agg_sh.at[pl.ds(s * RPT + done, n)])
            done += n
        plsc.subcore_barrier()

        def chunk(ci, carry):
            ebase = wid * EPW + ci * ch
            rbase = wid * (EPW // 128) + ci * nj
            pltpu.sync_copy(src_hbm.at[pl.ds(rbase, nj)], src_v)
            pltpu.sync_copy(dst_hbm.at[pl.ds(rbase, nj)], dst_v)
            pltpu.sync_copy(z_hbm.at[pl.ds(ebase, ch)], z_v)
            cps = [pltpu.async_copy(t_hbm.at[src_v.at[j]],
                                    rows_v.at[pl.ds(j * 128, 128)], sem)
                   for j in range(nj)]
            for cp in cps:
                cp.wait()

            def edge(e, carry2):
                zrow = z_v[e, pl.ds(zds, 16)]
                if accumulate:
                    acc = rows_v[e, pl.ds(n_k * n_o, 16)]
                    for j in range(n_k * n_o // 16):
                        segs = _zz_segments(j * 16, src_of)
                        zz = _build_zz(segs, zrow, lambda k: zoff + k)
                        acc = acc + rows_v[e, pl.ds(j * 16, 16)] * zz
                    msg_v[e, pl.ds(0, 16)] = acc
                else:
                    prods = []
                    for j in range(7):
                        segs = [(b, sp if sp != 0.0 else 1.0)
                                for b, sp in _zz_segments(j * 16, src_of)]
                        zz = _build_zz(segs, zrow, lambda k: zoff + k)
                        prods.append(rows_v[e, pl.ds(j * 16, 16)] * zz)
                    for j in range(5):
                        v = prods[j] + prods[j + 5] if j < 2 else prods[j]
                        msg_v[e, pl.ds(j * 16, 16)] = v
                return carry2

            lax.fori_loop(0, ch, edge, 0)
            for j in range(nj):
                pltpu.sync_copy(msg_v.at[pl.ds(j * 128, 128)],
                                agg_sh.at[dst_v.at[j]], add=True)
            return carry

        lax.fori_loop(0, nch, chunk, 0)
        plsc.subcore_barrier()
        pltpu.sync_copy(agg_sh.at[pl.ds(s * RPT, RPT)],
                        agg_hbm.at[c].at[pl.ds(s * RPT, RPT)])

    return sc_layer


def _make_sc_ep(ch):
    nch = EPW // ch
    nj = ch // 128
    mesh = plsc.VectorSubcoreMesh(core_axis_name="c", subcore_axis_name="s")

    @functools.partial(
        pl.kernel,
        out_type=jax.ShapeDtypeStruct((EPAD, 16), _f32),
        mesh=mesh,
        compiler_params=pltpu.CompilerParams(use_tc_tiling_on_sc=False),
        scratch_types=[
            pltpu.VMEM((nj, 128), jnp.int32),
            pltpu.VMEM((ch, 16), _f32),
            pltpu.VMEM((ch, 128), _f32),
            pltpu.VMEM((ch, 16), _f32),
            pltpu.SemaphoreType.DMA,
        ],
    )
    def sc_ep(q_hbm, a_hbm, src_hbm, out_hbm, src_v, a_v, q_v, r_v, sem):
        c = lax.axis_index("c")
        s = lax.axis_index("s")
        wid = c * NS + s

        def chunk(ci, carry):
            ebase = wid * EPW + ci * ch
            rbase = wid * (EPW // 128) + ci * nj
            pltpu.sync_copy(src_hbm.at[pl.ds(rbase, nj)], src_v)
            pltpu.sync_copy(a_hbm.at[pl.ds(ebase, ch)], a_v)
            cps = [pltpu.async_copy(q_hbm.at[src_v.at[j]],
                                    q_v.at[pl.ds(j * 128, 128)], sem)
                   for j in range(nj)]
            for cp in cps:
                cp.wait()

            def edge(e, carry2):
                v = a_v[e, pl.ds(0, 16)] + q_v[e, pl.ds(0, 16)]
                r_v[e, pl.ds(0, 16)] = jnp.maximum(v, 0.0)
                return carry2

            lax.fori_loop(0, ch, edge, 0)
            pltpu.sync_copy(r_v, out_hbm.at[pl.ds(ebase, ch)])
            return carry

        lax.fori_loop(0, nch, chunk, 0)

    return sc_ep


_sc_l0 = _make_sc_layer(ch=512, n_k=10, n_o=8, zds=0, zoff=0, mw=16,
                        accumulate=True)
_sc_l1 = _make_sc_layer(ch=256, n_k=10, n_o=10, zds=4, zoff=6, mw=80,
                        accumulate=False)
_sc_ep = _make_sc_ep(ch=512)


def kernel(x, edge_attr, edge_index, nn0_W1, nn0_b1, nn0_W2, nn0_b2, root0,
           bias0, nn1_W1, nn1_b1, nn1_W2, nn1_b2, root1, bias1, ep_W1, ep_b1,
           ep_W2, ep_b2):
    src = jnp.concatenate([edge_index[0], jnp.zeros((EPAD - E,), jnp.int32)])
    dst = jnp.concatenate([edge_index[1],
                           jnp.full((EPAD - E,), N, jnp.int32)])
    src2d = src.reshape(EPAD // 128, 128)
    dst2d = dst.reshape(EPAD // 128, 128)
    ea = jnp.concatenate([edge_attr, jnp.zeros((EPAD - E, DE), _f32)], axis=0)

    wz = jnp.concatenate([nn0_W1, nn1_W1, ep_W1[:DE]], axis=1)
    bz = jnp.concatenate([nn0_b1, nn1_b1, ep_b1]).reshape(1, 30)

    m0 = nn0_W2.reshape(IH, DF, H0).transpose(1, 0, 2).reshape(DF, IH * H0)
    w0 = jnp.concatenate([m0, nn0_b2.reshape(DF, H0), root0], axis=1)

    m1 = nn1_W2.reshape(IH, H0, IH).transpose(1, 0, 2).reshape(H0, IH * IH)
    w1 = jnp.concatenate([m1, nn1_b2.reshape(H0, IH), root1], axis=1)

    wq = jnp.concatenate(
        [jnp.concatenate([ep_W1[DE:], jnp.zeros((IH, 6), _f32)], axis=1),
         jnp.zeros((6, 16), _f32)], axis=0)
    w2row = jnp.concatenate([ep_W2[:, 0], jnp.zeros((6,), _f32)]).reshape(1, 16)

    z01, a_e = _edge_mm(ea, wz, bz)
    t0, xr0 = _node0(x, w0, bias0.reshape(1, 8))
    agg0 = _sc_l0(t0, z01, src2d, dst2d)
    t1, hr1 = _comb1(agg0, xr0, w1, bias1.reshape(1, 10))
    agg1 = _sc_l1(t1, z01, src2d, dst2d)
    q = _combq(agg1, hr1, wq)
    s2d = _sc_ep(q, a_e, src2d)
    out = _finish(s2d, w2row)
    return out[:E, 0] + ep_b2[0]


# speedup vs baseline: 2.0323x; 1.1605x over previous
import functools

import jax
import jax.numpy as jnp
from jax import lax
from jax.experimental import pallas as pl
from jax.experimental.pallas import tpu as pltpu
from jax.experimental.pallas import tpu_sc as plsc

N = 10000
E = 160000
DF = 128
DE = 16
H0 = 8
IH = 10

NC, NS = 2, 16
NW = NC * NS
EPW = 5120
EPAD = NW * EPW
NAG = 10112
RPT = NAG // NS

_f32 = jnp.float32


def _prep_edges_k(ea_ref, w_ref, b_ref, s0_ref, s1_ref, zx0_ref, zx1_ref, a_ref):
    y = jnp.dot(ea_ref[...], w_ref[...], preferred_element_type=_f32) + b_ref[...]
    be = y.shape[0]
    one = jnp.ones((be, 1), _f32)
    pad5 = jnp.zeros((be, 5), _f32)
    z0h = jnp.concatenate([jnp.maximum(y[:, :10], 0.0), one, pad5], axis=1)
    z1h = jnp.concatenate([jnp.maximum(y[:, 10:20], 0.0), one, pad5], axis=1)
    zx0_ref[...] = jnp.dot(z0h, s0_ref[...], preferred_element_type=_f32)
    zx1_ref[...] = jnp.dot(z1h, s1_ref[...], preferred_element_type=_f32)
    a_ref[...] = jnp.concatenate([y[:, 20:], jnp.zeros((be, 6), _f32)], axis=1)


def _node0_k(x_ref, w_ref, b_ref, t_ref, r_ref):
    y = jnp.dot(x_ref[...], w_ref[...], preferred_element_type=_f32)
    be = y.shape[0]
    t_ref[...] = jnp.concatenate([y[:, :88], jnp.zeros((be, 8), _f32)], axis=1)
    r_ref[...] = y[:, 88:] + b_ref[...]


def _comb1_k(agg_ref, xr_ref, w_ref, b_ref, t_ref, r_ref):
    a = agg_ref[0] + agg_ref[1]
    h1 = jnp.maximum(a[:, :8] + a[:, 8:] + xr_ref[...], 0.0)
    y = jnp.dot(h1, w_ref[...], preferred_element_type=_f32)
    be = y.shape[0]
    t_ref[...] = jnp.concatenate([y[:, :110], jnp.zeros((be, 2), _f32)], axis=1)
    r_ref[...] = y[:, 110:] + b_ref[...]


def _combq_k(agg_ref, hr_ref, w_ref, q_ref):
    m = agg_ref[0] + agg_ref[1]
    pre = hr_ref[...]
    for k in range(8):
        pre = pre + m[:, k * 10:k * 10 + 10]
    h2 = jnp.maximum(pre, 0.0)
    be = h2.shape[0]
    h2p = jnp.concatenate([h2, jnp.zeros((be, 6), _f32)], axis=1)
    q_ref[...] = jnp.dot(h2p, w_ref[...], preferred_element_type=_f32)


def _finish_k(s_ref, w_ref, o_ref):
    s = jnp.sum(s_ref[...] * w_ref[...], axis=1, keepdims=True)
    o_ref[...] = jnp.broadcast_to(s, s_ref.shape)


def _edge_mm(ea, w, b, s0, s1):
    be, grid = 2048, EPAD // 2048
    return pl.pallas_call(
        _prep_edges_k,
        grid=(grid,),
        in_specs=[
            pl.BlockSpec((be, DE), lambda i: (i, 0)),
            pl.BlockSpec((DE, 30), lambda i: (0, 0)),
            pl.BlockSpec((1, 30), lambda i: (0, 0)),
            pl.BlockSpec((16, 96), lambda i: (0, 0)),
            pl.BlockSpec((16, 112), lambda i: (0, 0)),
        ],
        out_specs=[
            pl.BlockSpec((be, 96), lambda i: (i, 0)),
            pl.BlockSpec((be, 112), lambda i: (i, 0)),
            pl.BlockSpec((be, 16), lambda i: (i, 0)),
        ],
        out_shape=[
            jax.ShapeDtypeStruct((EPAD, 96), _f32),
            jax.ShapeDtypeStruct((EPAD, 112), _f32),
            jax.ShapeDtypeStruct((EPAD, 16), _f32),
        ],
    )(ea, w, b, s0, s1)


def _node0(x, w, b):
    bn, grid = 1000, N // 1000
    return pl.pallas_call(
        _node0_k,
        grid=(grid,),
        in_specs=[
            pl.BlockSpec((bn, DF), lambda i: (i, 0)),
            pl.BlockSpec((DF, 96), lambda i: (0, 0)),
            pl.BlockSpec((1, 8), lambda i: (0, 0)),
        ],
        out_specs=[
            pl.BlockSpec((bn, 96), lambda i: (i, 0)),
            pl.BlockSpec((bn, 8), lambda i: (i, 0)),
        ],
        out_shape=[
            jax.ShapeDtypeStruct((N, 96), _f32),
            jax.ShapeDtypeStruct((N, 8), _f32),
        ],
    )(x, w, b)


def _comb1(agg, xr, w, b):
    bn, grid = 1000, N // 1000
    return pl.pallas_call(
        _comb1_k,
        grid=(grid,),
        in_specs=[
            pl.BlockSpec((2, bn, 16), lambda i: (0, i, 0)),
            pl.BlockSpec((bn, 8), lambda i: (i, 0)),
            pl.BlockSpec((8, 120), lambda i: (0, 0)),
            pl.BlockSpec((1, 10), lambda i: (0, 0)),
        ],
        out_specs=[
            pl.BlockSpec((bn, 112), lambda i: (i, 0)),
            pl.BlockSpec((bn, 10), lambda i: (i, 0)),
        ],
        out_shape=[
            jax.ShapeDtypeStruct((N, 112), _f32),
            jax.ShapeDtypeStruct((N, 10), _f32),
        ],
    )(agg, xr, w, b)


def _combq(agg, hr, w):
    bn, grid = 1000, N // 1000
    return pl.pallas_call(
        _combq_k,
        grid=(grid,),
        in_specs=[
            pl.BlockSpec((2, bn, 80), lambda i: (0, i, 0)),
            pl.BlockSpec((bn, 10), lambda i: (i, 0)),
            pl.BlockSpec((16, 16), lambda i: (0, 0)),
        ],
        out_specs=pl.BlockSpec((bn, 16), lambda i: (i, 0)),
        out_shape=jax.ShapeDtypeStruct((N, 16), _f32),
    )(agg, hr, w)


def _finish(s2d, w2row):
    be, grid = 2048, EPAD // 2048
    return pl.pallas_call(
        _finish_k,
        grid=(grid,),
        in_specs=[
            pl.BlockSpec((be, 16), lambda i: (i, 0)),
            pl.BlockSpec((1, 16), lambda i: (0, 0)),
        ],
        out_specs=pl.BlockSpec((be, 16), lambda i: (i, 0)),
        out_shape=jax.ShapeDtypeStruct((EPAD, 16), _f32),
    )(s2d, w2row)


def _lane():
    return lax.iota(jnp.int32, 16)


def _zz_segments(col_lo, src_of):
    segs = []
    for l in range(16):
        s = src_of(col_lo + l)
        if not segs or segs[-1][1] != s:
            segs.append((l, s))
    return segs


def _build_zz(segs, zrow, zlane_of_k):
    lane = _lane()

    def val(spec):
        if isinstance(spec, tuple):
            return jnp.full((16,), zrow[zlane_of_k(spec[1])], _f32)
        return jnp.full((16,), spec, _f32)

    zz = val(segs[0][1])
    for b, spec in segs[1:]:
        zz = jnp.where(lane >= b, val(spec), zz)
    return zz


def _make_sc_layer(ch, tw, mw, fold_pairs):
    nch = EPW // ch
    nj = ch // 128
    mesh = plsc.VectorSubcoreMesh(core_axis_name="c", subcore_axis_name="s")

    @functools.partial(
        pl.kernel,
        out_type=jax.ShapeDtypeStruct((NC, NAG, mw), _f32),
        mesh=mesh,
        compiler_params=pltpu.CompilerParams(use_tc_tiling_on_sc=False),
        scratch_types=[
            pltpu.VMEM((nj, 128), jnp.int32),
            pltpu.VMEM((nj, 128), jnp.int32),
            pltpu.VMEM((ch, tw), _f32),
            pltpu.VMEM((ch, tw), _f32),
            pltpu.VMEM((ch, mw), _f32),
            pltpu.VMEM_SHARED((NAG, mw), _f32),
            pltpu.SemaphoreType.DMA,
        ],
    )
    def sc_layer(t_hbm, zx_hbm, src_hbm, dst_hbm, agg_hbm,
                 src_v, dst_v, zx_v, rows_v, msg_v, agg_sh, sem):
        c = lax.axis_index("c")
        s = lax.axis_index("s")
        wid = c * NS + s

        @plsc.parallel_loop(0, ch, 1, unroll=8)
        def zrow_body(r):
            for j in range(mw // 16):
                msg_v[r, pl.ds(j * 16, 16)] = jnp.zeros((16,), _f32)

        done = 0
        while done < RPT:
            n = min(ch, RPT - done)
            pltpu.sync_copy(msg_v.at[pl.ds(0, n)],
                            agg_sh.at[pl.ds(s * RPT + done, n)])
            done += n
        plsc.subcore_barrier()

        def chunk(ci, carry):
            ebase = wid * EPW + ci * ch
            rbase = wid * (EPW // 128) + ci * nj
            pltpu.sync_copy(src_hbm.at[pl.ds(rbase, nj)], src_v)
            pltpu.sync_copy(dst_hbm.at[pl.ds(rbase, nj)], dst_v)
            pltpu.sync_copy(zx_hbm.at[pl.ds(ebase, ch)], zx_v)
            cps = [pltpu.async_copy(t_hbm.at[src_v.at[j]],
                                    rows_v.at[pl.ds(j * 128, 128)], sem)
                   for j in range(nj)]
            for cp in cps:
                cp.wait()

            @plsc.parallel_loop(0, ch, 1, unroll=4)
            def edge(e):
                if not fold_pairs:
                    acc = rows_v[e, pl.ds(0, 16)] * zx_v[e, pl.ds(0, 16)]
                    for j in range(1, tw // 16):
                        acc = acc + rows_v[e, pl.ds(j * 16, 16)] *                             zx_v[e, pl.ds(j * 16, 16)]
                    msg_v[e, pl.ds(0, 16)] = acc
                else:
                    prods = [rows_v[e, pl.ds(j * 16, 16)] *
                             zx_v[e, pl.ds(j * 16, 16)]
                             for j in range(tw // 16)]
                    for j in range(5):
                        v = prods[j] + prods[j + 5] if j < 2 else prods[j]
                        msg_v[e, pl.ds(j * 16, 16)] = v

            for j in range(nj):
                pltpu.sync_copy(msg_v.at[pl.ds(j * 128, 128)],
                                agg_sh.at[dst_v.at[j]], add=True)
            return carry

        lax.fori_loop(0, nch, chunk, 0)
        plsc.subcore_barrier()
        pltpu.sync_copy(agg_sh.at[pl.ds(s * RPT, RPT)],
                        agg_hbm.at[c].at[pl.ds(s * RPT, RPT)])

    return sc_layer


def _make_sc_ep(ch):
    nch = EPW // ch
    nj = ch // 128
    mesh = plsc.VectorSubcoreMesh(core_axis_name="c", subcore_axis_name="s")

    @functools.partial(
        pl.kernel,
        out_type=jax.ShapeDtypeStruct((EPAD, 16), _f32),
        mesh=mesh,
        compiler_params=pltpu.CompilerParams(use_tc_tiling_on_sc=False),
        scratch_types=[
            pltpu.VMEM((nj, 128), jnp.int32),
            pltpu.VMEM((ch, 16), _f32),
            pltpu.VMEM((ch, 16), _f32),
            pltpu.VMEM((ch, 16), _f32),
            pltpu.SemaphoreType.DMA,
        ],
    )
    def sc_ep(q_hbm, a_hbm, src_hbm, out_hbm, src_v, a_v, q_v, r_v, sem):
        c = lax.axis_index("c")
        s = lax.axis_index("s")
        wid = c * NS + s

        def chunk(ci, carry):
            ebase = wid * EPW + ci * ch
            rbase = wid * (EPW // 128) + ci * nj
            pltpu.sync_copy(src_hbm.at[pl.ds(rbase, nj)], src_v)
            pltpu.sync_copy(a_hbm.at[pl.ds(ebase, ch)], a_v)
            cps = [pltpu.async_copy(q_hbm.at[src_v.at[j]],
                                    q_v.at[pl.ds(j * 128, 128)], sem)
                   for j in range(nj)]
            for cp in cps:
                cp.wait()

            @plsc.parallel_loop(0, ch, 1, unroll=8)
            def edge(e):
                v = a_v[e, pl.ds(0, 16)] + q_v[e, pl.ds(0, 16)]
                r_v[e, pl.ds(0, 16)] = jnp.maximum(v, 0.0)

            pltpu.sync_copy(r_v, out_hbm.at[pl.ds(ebase, ch)])
            return carry

        lax.fori_loop(0, nch, chunk, 0)

    return sc_ep


_sc_l0 = _make_sc_layer(ch=512, tw=96, mw=16, fold_pairs=False)
_sc_l1 = _make_sc_layer(ch=256, tw=112, mw=80, fold_pairs=True)
_sc_ep = _make_sc_ep(ch=512)


def kernel(x, edge_attr, edge_index, nn0_W1, nn0_b1, nn0_W2, nn0_b2, root0,
           bias0, nn1_W1, nn1_b1, nn1_W2, nn1_b2, root1, bias1, ep_W1, ep_b1,
           ep_W2, ep_b2):
    src = jnp.concatenate([edge_index[0], jnp.zeros((EPAD - E,), jnp.int32)])
    dst = jnp.concatenate([edge_index[1],
                           jnp.full((EPAD - E,), N, jnp.int32)])
    src2d = src.reshape(EPAD // 128, 128)
    dst2d = dst.reshape(EPAD // 128, 128)
    ea = jnp.concatenate([edge_attr, jnp.zeros((EPAD - E, DE), _f32)], axis=0)

    wz = jnp.concatenate([nn0_W1, nn1_W1, ep_W1[:DE]], axis=1)
    bz = jnp.concatenate([nn0_b1, nn1_b1, ep_b1]).reshape(1, 30)

    k0 = jnp.arange(96) // 8
    s0 = (jnp.arange(16)[:, None] == jnp.where(k0 < 10, k0, 10)[None, :]
          ).astype(_f32) * (jnp.arange(96) < 88)[None, :]
    k1 = jnp.arange(112) // 10
    s1 = (jnp.arange(16)[:, None] == jnp.where(k1 < 10, k1, 10)[None, :]
          ).astype(_f32) * (jnp.arange(112) < 110)[None, :]

    m0 = nn0_W2.reshape(IH, DF, H0).transpose(1, 0, 2).reshape(DF, IH * H0)
    w0 = jnp.concatenate([m0, nn0_b2.reshape(DF, H0), root0], axis=1)

    m1 = nn1_W2.reshape(IH, H0, IH).transpose(1, 0, 2).reshape(H0, IH * IH)
    w1 = jnp.concatenate([m1, nn1_b2.reshape(H0, IH), root1], axis=1)

    wq = jnp.concatenate(
        [jnp.concatenate([ep_W1[DE:], jnp.zeros((IH, 6), _f32)], axis=1),
         jnp.zeros((6, 16), _f32)], axis=0)
    w2row = jnp.concatenate([ep_W2[:, 0], jnp.zeros((6,), _f32)]).reshape(1, 16)

    zx0, zx1, a_e = _edge_mm(ea, wz, bz, s0, s1)
    t0, xr0 = _node0(x, w0, bias0.reshape(1, 8))
    agg0 = _sc_l0(t0, zx0, src2d, dst2d)
    t1, hr1 = _comb1(agg0, xr0, w1, bias1.reshape(1, 10))
    agg1 = _sc_l1(t1, zx1, src2d, dst2d)
    q = _combq(agg1, hr1, wq)
    s2d = _sc_ep(q, a_e, src2d)
    out = _finish(s2d, w2row)
    return out[:E, 0] + ep_b2[0]
